# Initial kernel scaffold; baseline (speedup 1.0000x reference)
import jax
import jax.numpy as jnp
from jax.experimental import pallas as pl


def kernel(inputs, weight):
    raise NotImplementedError("write your pallas kernel here")


# speedup vs baseline: 1.6782x; 1.6782x over previous
import functools

import jax
import jax.numpy as jnp
from jax import lax
from jax.experimental import pallas as pl
from jax.experimental.pallas import tpu as pltpu
from jax.experimental.pallas import tpu_sc as plsc

NUM_CORES = 2
NUM_SUBCORES = 16
NUM_WORKERS = NUM_CORES * NUM_SUBCORES
CHUNK = 128


def _make_gather(dim, total_idx):
    assert total_idx % (CHUNK * NUM_WORKERS) == 0
    chunks_per_w = total_idx // (CHUNK * NUM_WORKERS)
    mesh = plsc.VectorSubcoreMesh(core_axis_name="c", subcore_axis_name="s")

    @functools.partial(
        pl.kernel,
        out_type=jax.ShapeDtypeStruct((total_idx, dim), jnp.float32),
        mesh=mesh,
        scratch_types=[
            pltpu.VMEM((chunks_per_w, CHUNK), jnp.int32),
            pltpu.VMEM((CHUNK, dim), jnp.float32),
            pltpu.SemaphoreType.DMA,
        ],
        compiler_params=pltpu.CompilerParams(use_tc_tiling_on_sc=False),
    )
    def gather_kernel(idx_hbm, table_hbm, out_hbm, idx_v, rows_v, gsem):
        wid = lax.axis_index("s") * NUM_CORES + lax.axis_index("c")
        chunk0 = wid * chunks_per_w
        pltpu.sync_copy(idx_hbm.at[pl.ds(chunk0, chunks_per_w)], idx_v)

        def body(j, _):
            pltpu.async_copy(table_hbm.at[idx_v.at[j]], rows_v, gsem).wait()
            row0 = (chunk0 + j) * CHUNK
            pltpu.sync_copy(rows_v, out_hbm.at[pl.ds(row0, CHUNK)])
            return 0

        lax.fori_loop(0, chunks_per_w, body, 0)

    return gather_kernel


def kernel(inputs, weight):
    original_shape = inputs.shape
    flat = inputs.reshape(-1).astype(jnp.int32)
    total = flat.shape[0]
    idx2d = flat.reshape(total // CHUNK, CHUNK)
    gather = _make_gather(weight.shape[1], total)
    out = gather(idx2d, weight)
    return out.reshape(original_shape + (weight.shape[1],))


# speedup vs baseline: 1.8673x; 1.1127x over previous
import functools

import jax
import jax.numpy as jnp
from jax import lax
from jax.experimental import pallas as pl
from jax.experimental.pallas import tpu as pltpu
from jax.experimental.pallas import tpu_sc as plsc

NUM_CORES = 2
NUM_SUBCORES = 16
NUM_WORKERS = NUM_CORES * NUM_SUBCORES
CHUNK = 128
K = 4


def _make_gather(dim, total_idx):
    assert total_idx % (CHUNK * NUM_WORKERS) == 0
    chunks_per_w = total_idx // (CHUNK * NUM_WORKERS)
    assert chunks_per_w % (2 * K) == 0
    num_groups = chunks_per_w // K
    mesh = plsc.VectorSubcoreMesh(core_axis_name="c", subcore_axis_name="s")

    @functools.partial(
        pl.kernel,
        out_type=jax.ShapeDtypeStruct((total_idx, dim), jnp.float32),
        mesh=mesh,
        scratch_types=[
            pltpu.VMEM((chunks_per_w, CHUNK), jnp.int32),
            pltpu.VMEM((2 * K, CHUNK, dim), jnp.float32),
            pltpu.SemaphoreType.DMA,
            pltpu.SemaphoreType.DMA,
            pltpu.SemaphoreType.DMA,
            pltpu.SemaphoreType.DMA,
        ],
        compiler_params=pltpu.CompilerParams(use_tc_tiling_on_sc=False),
    )
    def gather_kernel(
        idx_hbm, table_hbm, out_hbm, idx_v, rows_v, gsem0, gsem1, osem0, osem1
    ):
        gsem = (gsem0, gsem1)
        osem = (osem0, osem1)
        wid = lax.axis_index("s") * NUM_CORES + lax.axis_index("c")
        chunk0 = wid * chunks_per_w
        pltpu.sync_copy(idx_hbm.at[pl.ds(chunk0, chunks_per_w)], idx_v)

        def fire_gathers(g, p):
            for i in range(K):
                pltpu.async_copy(
                    table_hbm.at[idx_v.at[g * K + i]], rows_v.at[p * K + i], gsem[p]
                )

        def fire_wbs(g, p):
            for i in range(K):
                row0 = (chunk0 + g * K + i) * CHUNK
                pltpu.async_copy(
                    rows_v.at[p * K + i], out_hbm.at[pl.ds(row0, CHUNK)], osem[p]
                )

        def wait_gathers(p):
            for i in range(K):
                pltpu.make_async_copy(
                    table_hbm.at[idx_v.at[0]], rows_v.at[p * K + i], gsem[p]
                ).wait()

        def wait_wbs(p):
            for i in range(K):
                pltpu.make_async_copy(
                    rows_v.at[p * K + i], out_hbm.at[pl.ds(chunk0 * CHUNK, CHUNK)],
                    osem[p],
                ).wait()


        fire_gathers(0, 0)
        fire_gathers(1, 1)
        wait_gathers(0)
        fire_wbs(0, 0)
        wait_wbs(0)
        fire_gathers(2, 0)
        wait_gathers(1)
        fire_wbs(1, 1)

        def body(t, _):
            g1 = 2 * t + 1
            wait_wbs(1)
            fire_gathers(g1, 1)
            wait_gathers(0)
            fire_wbs(g1 - 1, 0)
            wait_wbs(0)

            @pl.when(g1 + 1 < num_groups)
            def _():
                fire_gathers(g1 + 1, 0)

            wait_gathers(1)
            fire_wbs(g1, 1)
            return 0

        lax.fori_loop(1, num_groups // 2, body, 0)
        wait_wbs(1)

    return gather_kernel


def kernel(inputs, weight):
    original_shape = inputs.shape
    flat = inputs.reshape(-1).astype(jnp.int32)
    total = flat.shape[0]
    idx2d = flat.reshape(total // CHUNK, CHUNK)
    gather = _make_gather(weight.shape[1], total)
    out = gather(idx2d, weight)
    return out.reshape(original_shape + (weight.shape[1],))
